# baseline (speedup 1.0000x reference)
import functools

import jax
import jax.numpy as jnp
from jax import lax
from jax.experimental import pallas as pl
from jax.experimental.pallas import tpu as pltpu
from jax.experimental.pallas import tpu_sc as plsc

N = 10000
D = 128
G = 64
N_PAD = 10112
STRIPE = N_PAD // 16
NW = 32
EB = 96
NB0 = 84
NB1 = 132
TBATCH = 16 * (NB0 + NB1)
E_TOT = 320000 + N
E_PAD = TBATCH * EB


def _tc_prologue_body(x_ref, w1_ref, b1_ref, w2_ref, b2_ref, wg_ref,
                      asrc_ref, adst_ref, h_ref, es_ref, ed_ref, c_ref):
    f = jnp.maximum(x_ref[...] @ w1_ref[...] + b1_ref[...][None, :], 0.0)
    f = jnp.maximum(f @ w2_ref[...] + b2_ref[...][None, :], 0.0)
    h = f @ wg_ref[...]
    h_ref[...] = h
    es = jnp.sum(h * asrc_ref[...][None, :], axis=1)
    ed = jnp.sum(h * adst_ref[...][None, :], axis=1)
    es_ref[...] = es
    ed_ref[...] = ed
    c0 = es + ed
    c_ref[...] = jnp.where(c0 > 0, c0, 0.2 * c0)


def _tc_mid_body(o_ref, s_ref, bg_ref, wg_ref, asrc_ref, adst_ref,
                 h_ref, es_ref, ed_ref, c_ref):
    s = s_ref[0, :] + s_ref[1, :] + 1e-16
    o = o_ref[0] + o_ref[1]
    f = jnp.maximum(o / s[:, None] + bg_ref[...][None, :], 0.0)
    h = f @ wg_ref[...]
    h_ref[...] = h
    es = jnp.sum(h * asrc_ref[...][None, :], axis=1)
    ed = jnp.sum(h * adst_ref[...][None, :], axis=1)
    es_ref[...] = es
    ed_ref[...] = ed
    c0 = es + ed
    c_ref[...] = jnp.where(c0 > 0, c0, 0.2 * c0)


def _tc_final_body(o_ref, s_ref, bg_ref, batch_ref, w3_ref, b3_ref, out_ref):
    s = s_ref[0, :] + s_ref[1, :] + 1e-16
    o = o_ref[0] + o_ref[1]
    f = jnp.maximum(o / s[:, None] + bg_ref[...][None, :], 0.0)
    b = batch_ref[...]
    gid = lax.broadcasted_iota(jnp.int32, (G, N_PAD), 0)
    onehot = (b[None, :] == gid).astype(jnp.float32)
    counts = jnp.sum(onehot, axis=1)
    sums = jnp.dot(onehot, f, preferred_element_type=jnp.float32)
    mean = sums / jnp.maximum(counts, 1.0)[:, None]
    out_ref[...] = jnp.sum(mean * w3_ref[...][:, 0][None, :], axis=1) + b3_ref[0]


def _tc_call(body, out_shapes, *args):
    return pl.pallas_call(
        body,
        out_shape=out_shapes,
    )(*args)


def _sc_edge_body(h_hbm, es_hbm, ed_hbm, c_hbm, srcs_hbm, dsts_hbm,
                  z2_hbm, z1_hbm,
                  out_hbm, s_hbm,
                  idxs_w0, idxd_w0, idxs_w1, idxd_w1,
                  esg_v0, edg_v0, cg_v0, ex_v0, rows_v0,
                  esg_v1, edg_v1, cg_v1, ex_v1, rows_v1,
                  s_stage_v,
                  out_acc, s_acc,
                  sem_i0, sem_i1,
                  sem_r0, sem_s0, sem_d0, sem_c0, sem_w0, sem_x0,
                  sem_r1, sem_s1, sem_d1, sem_c1, sem_w1, sem_x1):
    cid = lax.axis_index("c")
    sid = lax.axis_index("s")
    start = jnp.where(cid == 0, sid * NB0, 16 * NB0 + sid * NB1)
    nq = jnp.where(cid == 0, NB0 // 4, NB1 // 4)
    row0 = sid * STRIPE

    pltpu.sync_copy(z2_hbm.at[pl.ds(row0, STRIPE)],
                    out_acc.at[pl.ds(row0, STRIPE)])
    pltpu.sync_copy(z1_hbm.at[pl.ds(row0, STRIPE)], s_stage_v)
    pltpu.sync_copy(s_stage_v, s_acc.at[pl.ds(row0, STRIPE)])

    plsc.subcore_barrier()

    slots = (
        (esg_v0, edg_v0, cg_v0, ex_v0, rows_v0,
         sem_r0, sem_s0, sem_d0, sem_c0, sem_w0, sem_x0),
        (esg_v1, edg_v1, cg_v1, ex_v1, rows_v1,
         sem_r1, sem_s1, sem_d1, sem_c1, sem_w1, sem_x1),
    )
    windows = ((idxs_w0, idxd_w0, sem_i0), (idxs_w1, idxd_w1, sem_i1))

    def fetch_window(pair, win):
        idxs_w, idxd_w, sem_i = win
        pltpu.async_copy(srcs_hbm.at[pl.ds(start + pair * 2, 2)], idxs_w, sem_i)
        pltpu.async_copy(dsts_hbm.at[pl.ds(start + pair * 2, 2)], idxd_w, sem_i)

    def wait_row_scatter(slot, win):
        rows, sem_w = slot[4], slot[9]
        pltpu.make_async_copy(rows, out_acc.at[win[1].at[0, 0]], sem_w).wait()

    def wait_s_scatter(slot, win):
        ex, sem_x = slot[3], slot[10]
        pltpu.make_async_copy(ex, s_acc.at[win[1].at[0, 0]], sem_x).wait()

    def issue(slot, win, r, wait_idx, wait_ws):
        (esg, edg, cg, ex, rows,
         sem_r, sem_s, sem_d, sem_c, sem_w, sem_x) = slot
        idxs_w, idxd_w, sem_i = win
        if wait_idx:
            pltpu.make_async_copy(srcs_hbm.at[pl.ds(0, 2)],
                                  idxs_w, sem_i).wait()
            pltpu.make_async_copy(dsts_hbm.at[pl.ds(0, 2)],
                                  idxd_w, sem_i).wait()
        if wait_ws:
            wait_row_scatter(slot, win)
        src_row = idxs_w.at[r, 0]
        dst_row = idxd_w.at[r, 0]
        pltpu.async_copy(h_hbm.at[src_row], rows, sem_r)
        pltpu.async_copy(es_hbm.at[src_row], esg, sem_s)
        pltpu.async_copy(ed_hbm.at[dst_row], edg, sem_d)
        pltpu.async_copy(c_hbm.at[dst_row], cg, sem_c)

    def process(slot, win, r, wait_xs):
        (esg, edg, cg, ex, rows,
         sem_r, sem_s, sem_d, sem_c, sem_w, sem_x) = slot
        idxs_w, idxd_w, sem_i = win
        src_row = idxs_w.at[r, 0]
        dst_row = idxd_w.at[r, 0]
        pltpu.make_async_copy(es_hbm.at[src_row], esg, sem_s).wait()
        pltpu.make_async_copy(ed_hbm.at[dst_row], edg, sem_d).wait()
        pltpu.make_async_copy(c_hbm.at[dst_row], cg, sem_c).wait()
        if wait_xs:
            wait_s_scatter(slot, win)

        for k in range(EB // 16):
            ds16 = pl.ds(k * 16, 16)
            e = esg[ds16] + edg[ds16]
            e = jnp.where(e > 0, e, 0.2 * e)
            ex[ds16] = jnp.exp(e - cg[ds16])

        pltpu.async_copy(ex, s_acc.at[dst_row], sem_x, add=True)

        pltpu.make_async_copy(h_hbm.at[src_row], rows, sem_r).wait()

        def group_body(g, carry2):
            w16 = ex[pl.ds(g * 16, 16)]
            for i in range(16):
                w = w16[i]
                rr = g * 16 + i
                for kk in range(D // 16):
                    rows[rr, pl.ds(kk * 16, 16)] = (
                        rows[rr, pl.ds(kk * 16, 16)] * w)
            return carry2

        lax.fori_loop(0, EB // 16, group_body, 0, unroll=False)

        pltpu.async_copy(rows, out_acc.at[dst_row], sem_w, add=True)

    def quad(p, first):
        pair0 = p * 2

        issue(slots[1], windows[0], 1, wait_idx=False, wait_ws=not first)
        process(slots[0], windows[0], 0, wait_xs=not first)
        issue(slots[0], windows[1], 0, wait_idx=True, wait_ws=True)
        process(slots[1], windows[0], 1, wait_xs=not first)

        @pl.when(p + 1 < nq)
        def _():
            fetch_window(pair0 + 2, windows[0])

        issue(slots[1], windows[1], 1, wait_idx=False, wait_ws=True)
        process(slots[0], windows[1], 0, wait_xs=True)

        @pl.when(p + 1 < nq)
        def _():
            issue(slots[0], windows[0], 0, wait_idx=True, wait_ws=True)

        process(slots[1], windows[1], 1, wait_xs=True)

        @pl.when(p + 1 < nq)
        def _():
            fetch_window(pair0 + 3, windows[1])

        return 0

    fetch_window(0, windows[0])
    fetch_window(1, windows[1])
    issue(slots[0], windows[0], 0, wait_idx=True, wait_ws=False)

    quad(0, first=True)
    lax.fori_loop(1, nq, lambda p, c: quad(p, first=False), 0, unroll=False)

    for slot, win in zip(slots, windows):
        wait_row_scatter(slot, win)
        wait_s_scatter(slot, win)

    plsc.subcore_barrier()

    pltpu.sync_copy(out_acc.at[pl.ds(row0, STRIPE)],
                    out_hbm.at[cid, pl.ds(row0, STRIPE)])
    pltpu.sync_copy(s_acc.at[pl.ds(row0, STRIPE)], s_stage_v)
    pltpu.sync_copy(s_stage_v, s_hbm.at[cid, sid])


_sc_edge_layer = functools.partial(
    pl.kernel,
    out_type=[
        jax.ShapeDtypeStruct((2, N_PAD, D), jnp.float32),
        jax.ShapeDtypeStruct((2, 16, STRIPE), jnp.float32),
    ],
    scratch_types=(
        4 * [pltpu.VMEM((2, 1, EB), jnp.int32)]
        + 2 * [
            pltpu.VMEM((EB,), jnp.float32),
            pltpu.VMEM((EB,), jnp.float32),
            pltpu.VMEM((EB,), jnp.float32),
            pltpu.VMEM((EB,), jnp.float32),
            pltpu.VMEM((EB, D), jnp.float32),
        ]
        + [
            pltpu.VMEM((STRIPE,), jnp.float32),
            pltpu.VMEM_SHARED((N_PAD, D), jnp.float32),
            pltpu.VMEM_SHARED((N_PAD,), jnp.float32),
        ]
        + 14 * [pltpu.SemaphoreType.DMA]
    ),
    mesh=plsc.VectorSubcoreMesh(core_axis_name="c", subcore_axis_name="s"),
)(_sc_edge_body)


def kernel(x, edge_index, batch, W_lin1, b_lin1, W_lin2, b_lin2,
           Wg0, as0, ad0, bg0, Wg1, as1, ad1, bg1, Wg2, as2, ad2, bg2,
           W3, b3):
    loop = jnp.arange(N, dtype=edge_index.dtype)
    pad_e = E_PAD - E_TOT
    src = jnp.concatenate([edge_index[0], loop,
                           jnp.zeros((pad_e,), edge_index.dtype)])
    dst = jnp.concatenate([edge_index[1], loop,
                           N + jnp.arange(pad_e, dtype=edge_index.dtype) % 96])
    srcs = src.reshape(TBATCH, 1, EB)
    dsts = dst.reshape(TBATCH, 1, EB)

    x_pad = jnp.pad(x, ((0, N_PAD - N), (0, 0)))
    batch_pad = jnp.pad(batch, (0, N_PAD - N), constant_values=G)
    z2 = jnp.zeros((N_PAD, D), jnp.float32)
    z1 = jnp.zeros((N_PAD,), jnp.float32)

    node_shapes = [
        jax.ShapeDtypeStruct((N_PAD, D), jnp.float32),
        jax.ShapeDtypeStruct((N_PAD,), jnp.float32),
        jax.ShapeDtypeStruct((N_PAD,), jnp.float32),
        jax.ShapeDtypeStruct((N_PAD,), jnp.float32),
    ]

    h, es, ed, c = _tc_call(_tc_prologue_body, node_shapes,
                            x_pad, W_lin1, b_lin1, W_lin2, b_lin2,
                            Wg0, as0, ad0)

    for (asn, adn, wgn, bgp) in ((as1, ad1, Wg1, bg0), (as2, ad2, Wg2, bg1)):
        out_p, s_p = _sc_edge_layer(h, es, ed, c, srcs, dsts, z2, z1)
        h, es, ed, c = _tc_call(_tc_mid_body, node_shapes,
                                out_p, s_p.reshape(2, N_PAD), bgp, wgn, asn, adn)

    out_p, s_p = _sc_edge_layer(h, es, ed, c, srcs, dsts, z2, z1)
    final = _tc_call(_tc_final_body,
                     jax.ShapeDtypeStruct((G,), jnp.float32),
                     out_p, s_p.reshape(2, N_PAD), bg2, batch_pad, W3, b3)
    return final


# speedup vs baseline: 1.0707x; 1.0707x over previous
import functools

import jax
import jax.numpy as jnp
from jax import lax
from jax.experimental import pallas as pl
from jax.experimental.pallas import tpu as pltpu
from jax.experimental.pallas import tpu_sc as plsc

N = 10000
D = 128
G = 64
N_PAD = 10112
STRIPE = N_PAD // 16
NW = 32
EB = 96
NB0 = 108
NB1 = 108
TBATCH = 16 * (NB0 + NB1)
E_TOT = 320000 + N
E_PAD = TBATCH * EB


def _tc_prologue_body(x_ref, w1_ref, b1_ref, w2_ref, b2_ref, wg_ref,
                      asrc_ref, adst_ref, h_ref, es_ref, ed_ref, c_ref):
    f = jnp.maximum(x_ref[...] @ w1_ref[...] + b1_ref[...][None, :], 0.0)
    f = jnp.maximum(f @ w2_ref[...] + b2_ref[...][None, :], 0.0)
    h = f @ wg_ref[...]
    h_ref[...] = h
    es = jnp.sum(h * asrc_ref[...][None, :], axis=1)
    ed = jnp.sum(h * adst_ref[...][None, :], axis=1)
    es_ref[...] = es
    ed_ref[...] = ed
    c0 = es + ed
    c_ref[...] = jnp.where(c0 > 0, c0, 0.2 * c0)


def _tc_mid_body(o_ref, s_ref, bg_ref, wg_ref, asrc_ref, adst_ref,
                 h_ref, es_ref, ed_ref, c_ref):
    s = s_ref[0, :] + s_ref[1, :] + 1e-16
    o = o_ref[0] + o_ref[1]
    f = jnp.maximum(o / s[:, None] + bg_ref[...][None, :], 0.0)
    h = f @ wg_ref[...]
    h_ref[...] = h
    es = jnp.sum(h * asrc_ref[...][None, :], axis=1)
    ed = jnp.sum(h * adst_ref[...][None, :], axis=1)
    es_ref[...] = es
    ed_ref[...] = ed
    c0 = es + ed
    c_ref[...] = jnp.where(c0 > 0, c0, 0.2 * c0)


def _tc_final_body(o_ref, s_ref, bg_ref, batch_ref, w3_ref, b3_ref, out_ref):
    s = s_ref[0, :] + s_ref[1, :] + 1e-16
    o = o_ref[0] + o_ref[1]
    f = jnp.maximum(o / s[:, None] + bg_ref[...][None, :], 0.0)
    b = batch_ref[...]
    gid = lax.broadcasted_iota(jnp.int32, (G, N_PAD), 0)
    onehot = (b[None, :] == gid).astype(jnp.float32)
    counts = jnp.sum(onehot, axis=1)
    sums = jnp.dot(onehot, f, preferred_element_type=jnp.float32)
    mean = sums / jnp.maximum(counts, 1.0)[:, None]
    out_ref[...] = jnp.sum(mean * w3_ref[...][:, 0][None, :], axis=1) + b3_ref[0]


def _tc_call(body, out_shapes, *args):
    return pl.pallas_call(
        body,
        out_shape=out_shapes,
    )(*args)


def _sc_edge_body(h_hbm, es_hbm, ed_hbm, c_hbm, srcs_hbm, dsts_hbm,
                  z2_hbm, z1_hbm,
                  out_hbm, s_hbm,
                  idxs_w0, idxd_w0, idxs_w1, idxd_w1,
                  esg_v0, edg_v0, cg_v0, ex_v0, rows_v0,
                  esg_v1, edg_v1, cg_v1, ex_v1, rows_v1,
                  s_stage_v,
                  out_acc, s_acc,
                  sem_i0, sem_i1,
                  sem_r0, sem_s0, sem_d0, sem_c0, sem_w0, sem_x0,
                  sem_r1, sem_s1, sem_d1, sem_c1, sem_w1, sem_x1):
    cid = lax.axis_index("c")
    sid = lax.axis_index("s")
    start = jnp.where(cid == 0, sid * NB0, 16 * NB0 + sid * NB1)
    nq = jnp.where(cid == 0, NB0 // 4, NB1 // 4)
    row0 = sid * STRIPE

    pltpu.sync_copy(z2_hbm.at[pl.ds(row0, STRIPE)],
                    out_acc.at[pl.ds(row0, STRIPE)])
    pltpu.sync_copy(z1_hbm.at[pl.ds(row0, STRIPE)], s_stage_v)
    pltpu.sync_copy(s_stage_v, s_acc.at[pl.ds(row0, STRIPE)])

    plsc.subcore_barrier()

    slots = (
        (esg_v0, edg_v0, cg_v0, ex_v0, rows_v0,
         sem_r0, sem_s0, sem_d0, sem_c0, sem_w0, sem_x0),
        (esg_v1, edg_v1, cg_v1, ex_v1, rows_v1,
         sem_r1, sem_s1, sem_d1, sem_c1, sem_w1, sem_x1),
    )
    windows = ((idxs_w0, idxd_w0, sem_i0), (idxs_w1, idxd_w1, sem_i1))

    def fetch_window(pair, win):
        idxs_w, idxd_w, sem_i = win
        pltpu.async_copy(srcs_hbm.at[pl.ds(start + pair * 2, 2)], idxs_w, sem_i)
        pltpu.async_copy(dsts_hbm.at[pl.ds(start + pair * 2, 2)], idxd_w, sem_i)

    def wait_row_scatter(slot, win):
        rows, sem_w = slot[4], slot[9]
        pltpu.make_async_copy(rows, out_acc.at[win[1].at[0, 0]], sem_w).wait()

    def wait_s_scatter(slot, win):
        ex, sem_x = slot[3], slot[10]
        pltpu.make_async_copy(ex, s_acc.at[win[1].at[0, 0]], sem_x).wait()

    def issue(slot, win, r, wait_idx, wait_ws):
        (esg, edg, cg, ex, rows,
         sem_r, sem_s, sem_d, sem_c, sem_w, sem_x) = slot
        idxs_w, idxd_w, sem_i = win
        if wait_idx:
            pltpu.make_async_copy(srcs_hbm.at[pl.ds(0, 2)],
                                  idxs_w, sem_i).wait()
            pltpu.make_async_copy(dsts_hbm.at[pl.ds(0, 2)],
                                  idxd_w, sem_i).wait()
        if wait_ws:
            wait_row_scatter(slot, win)
        src_row = idxs_w.at[r, 0]
        dst_row = idxd_w.at[r, 0]
        pltpu.async_copy(h_hbm.at[src_row], rows, sem_r)
        pltpu.async_copy(es_hbm.at[src_row], esg, sem_s)
        pltpu.async_copy(ed_hbm.at[dst_row], edg, sem_d)
        pltpu.async_copy(c_hbm.at[dst_row], cg, sem_c)

    def process(slot, win, r, wait_xs):
        (esg, edg, cg, ex, rows,
         sem_r, sem_s, sem_d, sem_c, sem_w, sem_x) = slot
        idxs_w, idxd_w, sem_i = win
        src_row = idxs_w.at[r, 0]
        dst_row = idxd_w.at[r, 0]
        pltpu.make_async_copy(es_hbm.at[src_row], esg, sem_s).wait()
        pltpu.make_async_copy(ed_hbm.at[dst_row], edg, sem_d).wait()
        pltpu.make_async_copy(c_hbm.at[dst_row], cg, sem_c).wait()
        if wait_xs:
            wait_s_scatter(slot, win)

        for k in range(EB // 16):
            ds16 = pl.ds(k * 16, 16)
            e = esg[ds16] + edg[ds16]
            e = jnp.where(e > 0, e, 0.2 * e)
            ex[ds16] = jnp.exp(e - cg[ds16])

        pltpu.async_copy(ex, s_acc.at[dst_row], sem_x, add=True)

        pltpu.make_async_copy(h_hbm.at[src_row], rows, sem_r).wait()

        def group_body(g, carry2):
            w16 = ex[pl.ds(g * 16, 16)]
            for i in range(16):
                w = w16[i]
                rr = g * 16 + i
                for kk in range(D // 16):
                    rows[rr, pl.ds(kk * 16, 16)] = (
                        rows[rr, pl.ds(kk * 16, 16)] * w)
            return carry2

        lax.fori_loop(0, EB // 16, group_body, 0, unroll=False)

        pltpu.async_copy(rows, out_acc.at[dst_row], sem_w, add=True)

    def quad(p, first):
        pair0 = p * 2

        issue(slots[1], windows[0], 1, wait_idx=False, wait_ws=not first)
        process(slots[0], windows[0], 0, wait_xs=not first)
        issue(slots[0], windows[1], 0, wait_idx=True, wait_ws=True)
        process(slots[1], windows[0], 1, wait_xs=not first)

        @pl.when(p + 1 < nq)
        def _():
            fetch_window(pair0 + 2, windows[0])

        issue(slots[1], windows[1], 1, wait_idx=False, wait_ws=True)
        process(slots[0], windows[1], 0, wait_xs=True)

        @pl.when(p + 1 < nq)
        def _():
            issue(slots[0], windows[0], 0, wait_idx=True, wait_ws=True)

        process(slots[1], windows[1], 1, wait_xs=True)

        @pl.when(p + 1 < nq)
        def _():
            fetch_window(pair0 + 3, windows[1])

        return 0

    fetch_window(0, windows[0])
    fetch_window(1, windows[1])
    issue(slots[0], windows[0], 0, wait_idx=True, wait_ws=False)

    quad(0, first=True)
    lax.fori_loop(1, nq, lambda p, c: quad(p, first=False), 0, unroll=False)

    for slot, win in zip(slots, windows):
        wait_row_scatter(slot, win)
        wait_s_scatter(slot, win)

    plsc.subcore_barrier()

    pltpu.sync_copy(out_acc.at[pl.ds(row0, STRIPE)],
                    out_hbm.at[cid, pl.ds(row0, STRIPE)])
    pltpu.sync_copy(s_acc.at[pl.ds(row0, STRIPE)], s_stage_v)
    pltpu.sync_copy(s_stage_v, s_hbm.at[cid, sid])


_sc_edge_layer = functools.partial(
    pl.kernel,
    out_type=[
        jax.ShapeDtypeStruct((2, N_PAD, D), jnp.float32),
        jax.ShapeDtypeStruct((2, 16, STRIPE), jnp.float32),
    ],
    scratch_types=(
        4 * [pltpu.VMEM((2, 1, EB), jnp.int32)]
        + 2 * [
            pltpu.VMEM((EB,), jnp.float32),
            pltpu.VMEM((EB,), jnp.float32),
            pltpu.VMEM((EB,), jnp.float32),
            pltpu.VMEM((EB,), jnp.float32),
            pltpu.VMEM((EB, D), jnp.float32),
        ]
        + [
            pltpu.VMEM((STRIPE,), jnp.float32),
            pltpu.VMEM_SHARED((N_PAD, D), jnp.float32),
            pltpu.VMEM_SHARED((N_PAD,), jnp.float32),
        ]
        + 14 * [pltpu.SemaphoreType.DMA]
    ),
    mesh=plsc.VectorSubcoreMesh(core_axis_name="c", subcore_axis_name="s"),
)(_sc_edge_body)


def kernel(x, edge_index, batch, W_lin1, b_lin1, W_lin2, b_lin2,
           Wg0, as0, ad0, bg0, Wg1, as1, ad1, bg1, Wg2, as2, ad2, bg2,
           W3, b3):
    loop = jnp.arange(N, dtype=edge_index.dtype)
    pad_e = E_PAD - E_TOT
    src = jnp.concatenate([edge_index[0], loop,
                           jnp.zeros((pad_e,), edge_index.dtype)])
    dst = jnp.concatenate([edge_index[1], loop,
                           N + jnp.arange(pad_e, dtype=edge_index.dtype) % 96])
    srcs = src.reshape(TBATCH, 1, EB)
    dsts = dst.reshape(TBATCH, 1, EB)

    x_pad = jnp.pad(x, ((0, N_PAD - N), (0, 0)))
    batch_pad = jnp.pad(batch, (0, N_PAD - N), constant_values=G)
    z2 = jnp.zeros((N_PAD, D), jnp.float32)
    z1 = jnp.zeros((N_PAD,), jnp.float32)

    node_shapes = [
        jax.ShapeDtypeStruct((N_PAD, D), jnp.float32),
        jax.ShapeDtypeStruct((N_PAD,), jnp.float32),
        jax.ShapeDtypeStruct((N_PAD,), jnp.float32),
        jax.ShapeDtypeStruct((N_PAD,), jnp.float32),
    ]

    h, es, ed, c = _tc_call(_tc_prologue_body, node_shapes,
                            x_pad, W_lin1, b_lin1, W_lin2, b_lin2,
                            Wg0, as0, ad0)

    for (asn, adn, wgn, bgp) in ((as1, ad1, Wg1, bg0), (as2, ad2, Wg2, bg1)):
        out_p, s_p = _sc_edge_layer(h, es, ed, c, srcs, dsts, z2, z1)
        h, es, ed, c = _tc_call(_tc_mid_body, node_shapes,
                                out_p, s_p.reshape(2, N_PAD), bgp, wgn, asn, adn)

    out_p, s_p = _sc_edge_layer(h, es, ed, c, srcs, dsts, z2, z1)
    final = _tc_call(_tc_final_body,
                     jax.ShapeDtypeStruct((G,), jnp.float32),
                     out_p, s_p.reshape(2, N_PAD), bg2, batch_pad, W3, b3)
    return final
